# baseline (device time: 165547 ns/iter reference)
import jax
import jax.numpy as jnp
from jax import lax
from jax.experimental import pallas as pl
from jax.experimental.pallas import tpu as pltpu

N_DEV = 16
NQ = 4
NZ = 4
UPQ = 8
UH = UPQ // 2


def kernel(x):
    m, n = x.shape
    u_rows = m // (2 * NQ * UPQ)
    xr = x.reshape(2 * NQ * UPQ, u_rows, n).astype(jnp.bfloat16)

    def body(x_ref, out_ref, rbuf, obuf, zrbuf,
             p1s, p1r, p2rs, p2rr, p2bs, p2br, p3s, p3r):
        my = lax.axis_index("i")
        q = lax.rem(my, NQ)
        z = my // NQ
        pright = (my // NQ) * NQ + lax.rem(q + 1, NQ)
        pleft = (my // NQ) * NQ + lax.rem(q + NQ - 1, NQ)
        up = lax.min(my + NQ, N_DEV - 1)
        down = lax.max(my - NQ, 0)

        bar = pltpu.get_barrier_semaphore()
        for tgt in range(N_DEV):
            pl.semaphore_signal(
                bar, inc=1, device_id=(tgt,),
                device_id_type=pl.DeviceIdType.MESH,
            )
        pl.semaphore_wait(bar, N_DEV)

        peer = [pright, pleft]
        own = [lax.rem(q + 1, NQ), lax.rem(q + NQ - 1, NQ)]
        drain = []

        def rcopy(src, dst, ssem, rsem, dev):
            return pltpu.make_async_remote_copy(
                src_ref=src, dst_ref=dst, send_sem=ssem, recv_sem=rsem,
                device_id=(dev,), device_id_type=pl.DeviceIdType.MESH,
            )

        p1descs = [[[None] * UPQ for _ in range(NQ - 1)] for _ in range(2)]
        for s in range(NQ - 1):
            cs = [lax.rem(q - s + NQ, NQ), lax.rem(q + s, NQ)]
            for u in range(UPQ):
                for l in range(2):
                    uid = l * NQ * UPQ + cs[l] * UPQ + u
                    if s == 0:
                        src = x_ref.at[uid]
                    else:
                        p1descs[l][s - 1][u].wait_recv()
                        slot = l * (NQ - 1) * UPQ + (s - 1) * UPQ + u
                        rbuf[slot] = rbuf[slot] + x_ref[uid]
                        src = rbuf.at[slot]
                    d = rcopy(
                        src,
                        rbuf.at[l * (NQ - 1) * UPQ + s * UPQ + u],
                        p1s.at[l, s, u], p1r.at[l, s, u], peer[l],
                    )
                    d.start()
                    drain.append((d, None))
                    p1descs[l][s][u] = d

        p2r_descs = [[None] * UPQ for _ in range(2)]
        p2b_descs = [[None] * UPQ for _ in range(2)]
        for u in range(UPQ):
            for l in range(2):
                p1descs[l][NQ - 2][u].wait_recv()
                ob = l * UPQ + u
                uid_o = l * NQ * UPQ + own[l] * UPQ + u
                rslot = l * (NQ - 1) * UPQ + (NQ - 2) * UPQ + u
                obuf[ob] = rbuf[rslot] + x_ref[uid_o]

        for u in range(UH):
            for l in range(2):
                for uu, dev_r in ((u, up), (u + UH, down)):
                    ob = l * UPQ + uu
                    uid_o = l * NQ * UPQ + own[l] * UPQ + uu
                    p2r_descs[l][uu] = rcopy(
                        obuf.at[ob], zrbuf.at[ob],
                        p2rs.at[l, uu], p2rr.at[l, uu], dev_r)
                    dev_b = down if uu == u else up
                    p2b_descs[l][uu] = rcopy(
                        out_ref.at[uid_o], out_ref.at[uid_o],
                        p2bs.at[l, uu], p2br.at[l, uu], dev_b)
                    drain.append((p2r_descs[l][uu],
                                  (z < NZ - 1) if uu == u else (z > 0)))
                    drain.append((p2b_descs[l][uu],
                                  (z > 0) if uu == u else (z < NZ - 1)))

        @pl.when(z == 0)
        def _():
            for u in range(UH):
                for l in range(2):
                    p2r_descs[l][u].start()
        @pl.when(z == NZ - 1)
        def _():
            for u in range(UH):
                for l in range(2):
                    p2r_descs[l][u + UH].start()

        for u in range(UH):
            for l in range(2):
                rd = p2r_descs[l][u]
                ob = l * UPQ + u
                @pl.when(z > 0)
                def _():
                    rd.wait_recv()
                    obuf[ob] = obuf[ob] + zrbuf[ob]
                @pl.when((z > 0) & (z < NZ - 1))
                def _():
                    rd.start()

        for u in range(UH):
            for l in range(2):
                rd = p2r_descs[l][u + UH]
                ob = l * UPQ + u + UH
                @pl.when(z < NZ - 1)
                def _():
                    rd.wait_recv()
                    obuf[ob] = obuf[ob] + zrbuf[ob]
                @pl.when((z > 0) & (z < NZ - 1))
                def _():
                    rd.start()

        for u in range(UH):
            for l in range(2):
                for uu, at_end in ((u, z == NZ - 1), (u + UH, z == 0)):
                    ob = l * UPQ + uu
                    uid_o = l * NQ * UPQ + own[l] * UPQ + uu
                    bd = p2b_descs[l][uu]
                    @pl.when(at_end)
                    def _():
                        out_ref[uid_o] = obuf[ob]
                        bd.start()

        p3descs = [[[None] * UPQ for _ in range(NQ - 1)] for _ in range(2)]
        for u in range(UH):
            for l in range(2):
                for uu, at_end in ((u, z == NZ - 1), (u + UH, z == 0)):
                    bd = p2b_descs[l][uu]
                    @pl.when(jnp.logical_not(at_end))
                    def _():
                        bd.wait_recv()
                    @pl.when((z > 0) & (z < NZ - 1))
                    def _():
                        bd.start()
            for l in range(2):
                for uu in (u, u + UH):
                    uid_o = l * NQ * UPQ + own[l] * UPQ + uu
                    d = rcopy(out_ref.at[uid_o], out_ref.at[uid_o],
                              p3s.at[l, 0, uu], p3r.at[l, 0, uu], peer[l])
                    d.start()
                    drain.append((d, None))
                    p3descs[l][0][uu] = d

        for t in range(1, NQ - 1):
            for u in range(UPQ):
                for l in range(2):
                    p3descs[l][t - 1][u].wait_recv()
                    c = [lax.rem(q + 1 - t + NQ, NQ),
                         lax.rem(q + NQ - 1 + t, NQ)][l]
                    uid = l * NQ * UPQ + c * UPQ + u
                    d = rcopy(out_ref.at[uid], out_ref.at[uid],
                              p3s.at[l, t, u], p3r.at[l, t, u], peer[l])
                    d.start()
                    drain.append((d, None))
                    p3descs[l][t][u] = d
        for u in range(UPQ):
            for l in range(2):
                p3descs[l][NQ - 2][u].wait_recv()

        for d, cond in drain:
            if cond is None:
                d.wait_send()
            else:
                @pl.when(cond)
                def _():
                    d.wait_send()

    nunit = 2 * NQ * UPQ
    out = pl.pallas_call(
        body,
        out_shape=jax.ShapeDtypeStruct((nunit, u_rows, n), jnp.bfloat16),
        in_specs=[pl.BlockSpec(memory_space=pltpu.VMEM)],
        out_specs=pl.BlockSpec(memory_space=pltpu.VMEM),
        scratch_shapes=[
            pltpu.VMEM((2 * (NQ - 1) * UPQ, u_rows, n), jnp.bfloat16),
            pltpu.VMEM((2 * UPQ, u_rows, n), jnp.bfloat16),
            pltpu.VMEM((2 * UPQ, u_rows, n), jnp.bfloat16),
            pltpu.SemaphoreType.DMA((2, NQ - 1, UPQ)),
            pltpu.SemaphoreType.DMA((2, NQ - 1, UPQ)),
            pltpu.SemaphoreType.DMA((2, UPQ)),
            pltpu.SemaphoreType.DMA((2, UPQ)),
            pltpu.SemaphoreType.DMA((2, UPQ)),
            pltpu.SemaphoreType.DMA((2, UPQ)),
            pltpu.SemaphoreType.DMA((2, NQ - 1, UPQ)),
            pltpu.SemaphoreType.DMA((2, NQ - 1, UPQ)),
        ],
        compiler_params=pltpu.CompilerParams(collective_id=0),
    )(xr)
    return out.reshape(m, n).astype(jnp.float32)


# device time: 148730 ns/iter; 1.1131x vs baseline; 1.1131x over previous
import jax
import jax.numpy as jnp
from jax import lax
from jax.experimental import pallas as pl
from jax.experimental.pallas import tpu as pltpu

N_DEV = 16
NQ = 4
NZ = 4
UPQ = 8


def kernel(x):
    m, n = x.shape
    u_rows = m // (2 * NQ * UPQ)
    xr = x.reshape(2 * NQ * UPQ, u_rows, n)

    def body(x_ref, out_ref, sbuf, rbuf, obuf, zrbuf,
             p1s, p1r, p2rs, p2rr, p2bs, p2br, p3s, p3r):
        my = lax.axis_index("i")
        q = lax.rem(my, NQ)
        z = my // NQ
        base = (my // NQ) * NQ
        pright = base + lax.rem(q + 1, NQ)
        pleft = base + lax.rem(q + NQ - 1, NQ)
        up = jnp.where(z < NZ - 1, my + NQ, my)
        down = jnp.where(z > 0, my - NQ, my)

        bar = pltpu.get_barrier_semaphore()
        for tgt in range(N_DEV):
            pl.semaphore_signal(
                bar, inc=1, device_id=(tgt,),
                device_id_type=pl.DeviceIdType.MESH,
            )
        pl.semaphore_wait(bar, N_DEV)

        peer = [pright, pleft]
        own = [lax.rem(q + 1, NQ), lax.rem(q + NQ - 1, NQ)]
        cast = lambda v: v.astype(jnp.bfloat16)

        def rcopy(src, dst, ssem, rsem, dev):
            return pltpu.make_async_remote_copy(
                src_ref=src, dst_ref=dst, send_sem=ssem, recv_sem=rsem,
                device_id=(dev,), device_id_type=pl.DeviceIdType.MESH,
            )

        p1descs = [[[None] * UPQ for _ in range(NQ - 1)] for _ in range(2)]
        for s in range(NQ - 1):
            cs = [lax.rem(q - s + NQ, NQ), lax.rem(q + s, NQ)]
            for u in range(UPQ):
                for l in range(2):
                    uid = l * NQ * UPQ + cs[l] * UPQ + u
                    if s == 0:
                        si = l * UPQ + u
                        sbuf[si] = cast(x_ref[uid])
                        src = sbuf.at[si]
                    else:
                        p1descs[l][s - 1][u].wait_recv()
                        slot = l * (NQ - 1) * UPQ + (s - 1) * UPQ + u
                        rbuf[slot] = rbuf[slot] + cast(x_ref[uid])
                        src = rbuf.at[slot]
                    d = rcopy(
                        src,
                        rbuf.at[l * (NQ - 1) * UPQ + s * UPQ + u],
                        p1s.at[l, s, u], p1r.at[l, s, u], peer[l],
                    )
                    d.start()
                    p1descs[l][s][u] = d

        p2r_descs = [[None] * UPQ for _ in range(2)]
        p2b_descs = [[None] * UPQ for _ in range(2)]
        for u in range(UPQ):
            flow_up = (u % 2 == 0)
            for l in range(2):
                ob = l * UPQ + u
                uid_o = l * NQ * UPQ + own[l] * UPQ + u
                p2r_descs[l][u] = rcopy(
                    obuf.at[ob], zrbuf.at[ob],
                    p2rs.at[l, u], p2rr.at[l, u],
                    up if flow_up else down)
                p2b_descs[l][u] = rcopy(
                    out_ref.at[uid_o], out_ref.at[uid_o],
                    p2bs.at[l, u], p2br.at[l, u],
                    down if flow_up else up)

        for u in range(UPQ):
            flow_up = (u % 2 == 0)
            for l in range(2):
                p1descs[l][NQ - 2][u].wait_recv()
                ob = l * UPQ + u
                uid_o = l * NQ * UPQ + own[l] * UPQ + u
                rslot = l * (NQ - 1) * UPQ + (NQ - 2) * UPQ + u
                obuf[ob] = rbuf[rslot] + cast(x_ref[uid_o])
                rd = p2r_descs[l][u]
                @pl.when((z == 0) if flow_up else (z == NZ - 1))
                def _():
                    rd.start()

        mid = (z > 0) & (z < NZ - 1)
        for u in range(UPQ):
            flow_up = (u % 2 == 0)
            for l in range(2):
                rd = p2r_descs[l][u]
                bd = p2b_descs[l][u]
                ob = l * UPQ + u
                uid_o = l * NQ * UPQ + own[l] * UPQ + u
                @pl.when((z > 0) if flow_up else (z < NZ - 1))
                def _():
                    rd.wait_recv()
                    obuf[ob] = obuf[ob] + zrbuf[ob]
                @pl.when(mid)
                def _():
                    rd.start()
                @pl.when((z == NZ - 1) if flow_up else (z == 0))
                def _():
                    out_ref[uid_o] = obuf[ob]
                    bd.start()

        p3descs = [[[None] * UPQ for _ in range(NQ - 1)] for _ in range(2)]
        for u in range(UPQ):
            flow_up = (u % 2 == 0)
            for l in range(2):
                bd = p2b_descs[l][u]
                @pl.when((z < NZ - 1) if flow_up else (z > 0))
                def _():
                    bd.wait_recv()
                @pl.when(mid)
                def _():
                    bd.start()
            for l in range(2):
                uid_o = l * NQ * UPQ + own[l] * UPQ + u
                d = rcopy(out_ref.at[uid_o], out_ref.at[uid_o],
                          p3s.at[l, 0, u], p3r.at[l, 0, u], peer[l])
                d.start()
                p3descs[l][0][u] = d

        for t in range(1, NQ - 1):
            for u in range(UPQ):
                for l in range(2):
                    p3descs[l][t - 1][u].wait_recv()
                    c = [lax.rem(q + 1 - t + NQ, NQ),
                         lax.rem(q + NQ - 1 + t, NQ)][l]
                    uid = l * NQ * UPQ + c * UPQ + u
                    d = rcopy(out_ref.at[uid], out_ref.at[uid],
                              p3s.at[l, t, u], p3r.at[l, t, u], peer[l])
                    d.start()
                    p3descs[l][t][u] = d
        for u in range(UPQ):
            for l in range(2):
                p3descs[l][NQ - 2][u].wait_recv()

        for l in range(2):
            for s in range(NQ - 1):
                for u in range(UPQ):
                    p1descs[l][s][u].wait_send()
                    p3descs[l][s][u].wait_send()
        @pl.when(z < NZ - 1)
        def _():
            for l in range(2):
                for u in range(UPQ):
                    if u % 2 == 0:
                        p2r_descs[l][u].wait_send()
                    else:
                        p2b_descs[l][u].wait_send()
        @pl.when(z > 0)
        def _():
            for l in range(2):
                for u in range(UPQ):
                    if u % 2 == 0:
                        p2b_descs[l][u].wait_send()
                    else:
                        p2r_descs[l][u].wait_send()

    nunit = 2 * NQ * UPQ
    out = pl.pallas_call(
        body,
        out_shape=jax.ShapeDtypeStruct((nunit, u_rows, n), jnp.bfloat16),
        in_specs=[pl.BlockSpec(memory_space=pltpu.VMEM)],
        out_specs=pl.BlockSpec(memory_space=pltpu.VMEM),
        scratch_shapes=[
            pltpu.VMEM((2 * UPQ, u_rows, n), jnp.bfloat16),
            pltpu.VMEM((2 * (NQ - 1) * UPQ, u_rows, n), jnp.bfloat16),
            pltpu.VMEM((2 * UPQ, u_rows, n), jnp.bfloat16),
            pltpu.VMEM((2 * UPQ, u_rows, n), jnp.bfloat16),
            pltpu.SemaphoreType.DMA((2, NQ - 1, UPQ)),
            pltpu.SemaphoreType.DMA((2, NQ - 1, UPQ)),
            pltpu.SemaphoreType.DMA((2, UPQ)),
            pltpu.SemaphoreType.DMA((2, UPQ)),
            pltpu.SemaphoreType.DMA((2, UPQ)),
            pltpu.SemaphoreType.DMA((2, UPQ)),
            pltpu.SemaphoreType.DMA((2, NQ - 1, UPQ)),
            pltpu.SemaphoreType.DMA((2, NQ - 1, UPQ)),
        ],
        compiler_params=pltpu.CompilerParams(collective_id=0),
    )(xr)
    return out.reshape(m, n)


# device time: 135409 ns/iter; 1.2226x vs baseline; 1.0984x over previous
import jax
import jax.numpy as jnp
from jax import lax
from jax.experimental import pallas as pl
from jax.experimental.pallas import tpu as pltpu

N_DEV = 16
NQ = 4
NZ = 4
UPQ = 4


def kernel(x):
    m, n = x.shape
    u_rows = m // (2 * NQ * UPQ)
    xr = x.reshape(2 * NQ * UPQ, u_rows, n)

    def body(x_ref, out_ref, sbuf, rbuf, obuf, zrbuf,
             p1s, p1r, p2rs, p2rr, p2bs, p2br, p3s, p3r):
        my = lax.axis_index("i")
        q = lax.rem(my, NQ)
        z = my // NQ
        base = (my // NQ) * NQ
        pright = base + lax.rem(q + 1, NQ)
        pleft = base + lax.rem(q + NQ - 1, NQ)
        up = jnp.where(z < NZ - 1, my + NQ, my)
        down = jnp.where(z > 0, my - NQ, my)

        bar = pltpu.get_barrier_semaphore()
        for tgt in range(N_DEV):
            pl.semaphore_signal(
                bar, inc=1, device_id=(tgt,),
                device_id_type=pl.DeviceIdType.MESH,
            )
        pl.semaphore_wait(bar, N_DEV)

        peer = [pright, pleft]
        own = [lax.rem(q + 1, NQ), lax.rem(q + NQ - 1, NQ)]
        cast = lambda v: v.astype(jnp.bfloat16)

        def rcopy(src, dst, ssem, rsem, dev):
            return pltpu.make_async_remote_copy(
                src_ref=src, dst_ref=dst, send_sem=ssem, recv_sem=rsem,
                device_id=(dev,), device_id_type=pl.DeviceIdType.MESH,
            )

        p1descs = [[[None] * UPQ for _ in range(NQ - 1)] for _ in range(2)]
        for s in range(NQ - 1):
            cs = [lax.rem(q - s + NQ, NQ), lax.rem(q + s, NQ)]
            for u in range(UPQ):
                for l in range(2):
                    uid = l * NQ * UPQ + cs[l] * UPQ + u
                    if s == 0:
                        si = l * UPQ + u
                        sbuf[si] = cast(x_ref[uid])
                        src = sbuf.at[si]
                    else:
                        p1descs[l][s - 1][u].wait_recv()
                        slot = l * (NQ - 1) * UPQ + (s - 1) * UPQ + u
                        rbuf[slot] = rbuf[slot] + cast(x_ref[uid])
                        src = rbuf.at[slot]
                    d = rcopy(
                        src,
                        rbuf.at[l * (NQ - 1) * UPQ + s * UPQ + u],
                        p1s.at[l, s, u], p1r.at[l, s, u], peer[l],
                    )
                    d.start()
                    p1descs[l][s][u] = d

        p2r_descs = [[None] * UPQ for _ in range(2)]
        p2b_descs = [[None] * UPQ for _ in range(2)]
        for u in range(UPQ):
            flow_up = (u % 2 == 0)
            for l in range(2):
                ob = l * UPQ + u
                uid_o = l * NQ * UPQ + own[l] * UPQ + u
                p2r_descs[l][u] = rcopy(
                    obuf.at[ob], zrbuf.at[ob],
                    p2rs.at[l, u], p2rr.at[l, u],
                    up if flow_up else down)
                p2b_descs[l][u] = rcopy(
                    out_ref.at[uid_o], out_ref.at[uid_o],
                    p2bs.at[l, u], p2br.at[l, u],
                    down if flow_up else up)

        for u in range(UPQ):
            flow_up = (u % 2 == 0)
            for l in range(2):
                p1descs[l][NQ - 2][u].wait_recv()
                ob = l * UPQ + u
                uid_o = l * NQ * UPQ + own[l] * UPQ + u
                rslot = l * (NQ - 1) * UPQ + (NQ - 2) * UPQ + u
                obuf[ob] = rbuf[rslot] + cast(x_ref[uid_o])
                rd = p2r_descs[l][u]
                @pl.when((z == 0) if flow_up else (z == NZ - 1))
                def _():
                    rd.start()

        mid = (z > 0) & (z < NZ - 1)
        for u in range(UPQ):
            flow_up = (u % 2 == 0)
            for l in range(2):
                rd = p2r_descs[l][u]
                bd = p2b_descs[l][u]
                ob = l * UPQ + u
                uid_o = l * NQ * UPQ + own[l] * UPQ + u
                @pl.when((z > 0) if flow_up else (z < NZ - 1))
                def _():
                    rd.wait_recv()
                    obuf[ob] = obuf[ob] + zrbuf[ob]
                @pl.when(mid)
                def _():
                    rd.start()
                @pl.when((z == NZ - 1) if flow_up else (z == 0))
                def _():
                    out_ref[uid_o] = obuf[ob]
                    bd.start()

        p3descs = [[[None] * UPQ for _ in range(NQ - 1)] for _ in range(2)]
        for u in range(UPQ):
            flow_up = (u % 2 == 0)
            for l in range(2):
                bd = p2b_descs[l][u]
                @pl.when((z < NZ - 1) if flow_up else (z > 0))
                def _():
                    bd.wait_recv()
                @pl.when(mid)
                def _():
                    bd.start()
            for l in range(2):
                uid_o = l * NQ * UPQ + own[l] * UPQ + u
                d = rcopy(out_ref.at[uid_o], out_ref.at[uid_o],
                          p3s.at[l, 0, u], p3r.at[l, 0, u], peer[l])
                d.start()
                p3descs[l][0][u] = d

        for t in range(1, NQ - 1):
            for u in range(UPQ):
                for l in range(2):
                    p3descs[l][t - 1][u].wait_recv()
                    c = [lax.rem(q + 1 - t + NQ, NQ),
                         lax.rem(q + NQ - 1 + t, NQ)][l]
                    uid = l * NQ * UPQ + c * UPQ + u
                    d = rcopy(out_ref.at[uid], out_ref.at[uid],
                              p3s.at[l, t, u], p3r.at[l, t, u], peer[l])
                    d.start()
                    p3descs[l][t][u] = d
        for u in range(UPQ):
            for l in range(2):
                p3descs[l][NQ - 2][u].wait_recv()

        for l in range(2):
            for s in range(NQ - 1):
                for u in range(UPQ):
                    p1descs[l][s][u].wait_send()
                    p3descs[l][s][u].wait_send()
        @pl.when(z < NZ - 1)
        def _():
            for l in range(2):
                for u in range(UPQ):
                    if u % 2 == 0:
                        p2r_descs[l][u].wait_send()
                    else:
                        p2b_descs[l][u].wait_send()
        @pl.when(z > 0)
        def _():
            for l in range(2):
                for u in range(UPQ):
                    if u % 2 == 0:
                        p2b_descs[l][u].wait_send()
                    else:
                        p2r_descs[l][u].wait_send()

    nunit = 2 * NQ * UPQ
    out = pl.pallas_call(
        body,
        out_shape=jax.ShapeDtypeStruct((nunit, u_rows, n), jnp.bfloat16),
        in_specs=[pl.BlockSpec(memory_space=pltpu.VMEM)],
        out_specs=pl.BlockSpec(memory_space=pltpu.VMEM),
        scratch_shapes=[
            pltpu.VMEM((2 * UPQ, u_rows, n), jnp.bfloat16),
            pltpu.VMEM((2 * (NQ - 1) * UPQ, u_rows, n), jnp.bfloat16),
            pltpu.VMEM((2 * UPQ, u_rows, n), jnp.bfloat16),
            pltpu.VMEM((2 * UPQ, u_rows, n), jnp.bfloat16),
            pltpu.SemaphoreType.DMA((2, NQ - 1, UPQ)),
            pltpu.SemaphoreType.DMA((2, NQ - 1, UPQ)),
            pltpu.SemaphoreType.DMA((2, UPQ)),
            pltpu.SemaphoreType.DMA((2, UPQ)),
            pltpu.SemaphoreType.DMA((2, UPQ)),
            pltpu.SemaphoreType.DMA((2, UPQ)),
            pltpu.SemaphoreType.DMA((2, NQ - 1, UPQ)),
            pltpu.SemaphoreType.DMA((2, NQ - 1, UPQ)),
        ],
        compiler_params=pltpu.CompilerParams(collective_id=0),
    )(xr)
    return out.reshape(m, n)


# device time: 133035 ns/iter; 1.2444x vs baseline; 1.0178x over previous
import jax
import jax.numpy as jnp
from jax import lax
from jax.experimental import pallas as pl
from jax.experimental.pallas import tpu as pltpu

N_DEV = 16
NQ = 4
NZ = 4
UPQ = 2


def kernel(x):
    m, n = x.shape
    u_rows = m // (2 * NQ * UPQ)
    xr = x.reshape(2 * NQ * UPQ, u_rows, n)

    def body(x_ref, out_ref, sbuf, rbuf, obuf, zrbuf,
             p1s, p1r, p2rs, p2rr, p2bs, p2br, p3s, p3r):
        my = lax.axis_index("i")
        q = lax.rem(my, NQ)
        z = my // NQ
        base = (my // NQ) * NQ
        pright = base + lax.rem(q + 1, NQ)
        pleft = base + lax.rem(q + NQ - 1, NQ)
        up = jnp.where(z < NZ - 1, my + NQ, my)
        down = jnp.where(z > 0, my - NQ, my)

        bar = pltpu.get_barrier_semaphore()
        for tgt in range(N_DEV):
            pl.semaphore_signal(
                bar, inc=1, device_id=(tgt,),
                device_id_type=pl.DeviceIdType.MESH,
            )
        pl.semaphore_wait(bar, N_DEV)

        peer = [pright, pleft]
        own = [lax.rem(q + 1, NQ), lax.rem(q + NQ - 1, NQ)]
        cast = lambda v: v.astype(jnp.bfloat16)

        def rcopy(src, dst, ssem, rsem, dev):
            return pltpu.make_async_remote_copy(
                src_ref=src, dst_ref=dst, send_sem=ssem, recv_sem=rsem,
                device_id=(dev,), device_id_type=pl.DeviceIdType.MESH,
            )

        p1descs = [[[None] * UPQ for _ in range(NQ - 1)] for _ in range(2)]
        for s in range(NQ - 1):
            cs = [lax.rem(q - s + NQ, NQ), lax.rem(q + s, NQ)]
            for u in range(UPQ):
                for l in range(2):
                    uid = l * NQ * UPQ + cs[l] * UPQ + u
                    if s == 0:
                        si = l * UPQ + u
                        sbuf[si] = cast(x_ref[uid])
                        src = sbuf.at[si]
                    else:
                        p1descs[l][s - 1][u].wait_recv()
                        slot = l * (NQ - 1) * UPQ + (s - 1) * UPQ + u
                        rbuf[slot] = rbuf[slot] + cast(x_ref[uid])
                        src = rbuf.at[slot]
                    d = rcopy(
                        src,
                        rbuf.at[l * (NQ - 1) * UPQ + s * UPQ + u],
                        p1s.at[l, s, u], p1r.at[l, s, u], peer[l],
                    )
                    d.start()
                    p1descs[l][s][u] = d

        p2r_descs = [[None] * UPQ for _ in range(2)]
        p2b_descs = [[None] * UPQ for _ in range(2)]
        for u in range(UPQ):
            flow_up = (u % 2 == 0)
            for l in range(2):
                ob = l * UPQ + u
                uid_o = l * NQ * UPQ + own[l] * UPQ + u
                p2r_descs[l][u] = rcopy(
                    obuf.at[ob], zrbuf.at[ob],
                    p2rs.at[l, u], p2rr.at[l, u],
                    up if flow_up else down)
                p2b_descs[l][u] = rcopy(
                    out_ref.at[uid_o], out_ref.at[uid_o],
                    p2bs.at[l, u], p2br.at[l, u],
                    down if flow_up else up)

        for u in range(UPQ):
            flow_up = (u % 2 == 0)
            for l in range(2):
                p1descs[l][NQ - 2][u].wait_recv()
                ob = l * UPQ + u
                uid_o = l * NQ * UPQ + own[l] * UPQ + u
                rslot = l * (NQ - 1) * UPQ + (NQ - 2) * UPQ + u
                obuf[ob] = rbuf[rslot] + cast(x_ref[uid_o])
                rd = p2r_descs[l][u]
                @pl.when((z == 0) if flow_up else (z == NZ - 1))
                def _():
                    rd.start()

        mid = (z > 0) & (z < NZ - 1)
        for u in range(UPQ):
            flow_up = (u % 2 == 0)
            for l in range(2):
                rd = p2r_descs[l][u]
                bd = p2b_descs[l][u]
                ob = l * UPQ + u
                uid_o = l * NQ * UPQ + own[l] * UPQ + u
                @pl.when((z > 0) if flow_up else (z < NZ - 1))
                def _():
                    rd.wait_recv()
                    obuf[ob] = obuf[ob] + zrbuf[ob]
                @pl.when(mid)
                def _():
                    rd.start()
                @pl.when((z == NZ - 1) if flow_up else (z == 0))
                def _():
                    out_ref[uid_o] = obuf[ob]
                    bd.start()

        p3descs = [[[None] * UPQ for _ in range(NQ - 1)] for _ in range(2)]
        for u in range(UPQ):
            flow_up = (u % 2 == 0)
            for l in range(2):
                bd = p2b_descs[l][u]
                @pl.when((z < NZ - 1) if flow_up else (z > 0))
                def _():
                    bd.wait_recv()
                @pl.when(mid)
                def _():
                    bd.start()
            for l in range(2):
                uid_o = l * NQ * UPQ + own[l] * UPQ + u
                d = rcopy(out_ref.at[uid_o], out_ref.at[uid_o],
                          p3s.at[l, 0, u], p3r.at[l, 0, u], peer[l])
                d.start()
                p3descs[l][0][u] = d

        for t in range(1, NQ - 1):
            for u in range(UPQ):
                for l in range(2):
                    p3descs[l][t - 1][u].wait_recv()
                    c = [lax.rem(q + 1 - t + NQ, NQ),
                         lax.rem(q + NQ - 1 + t, NQ)][l]
                    uid = l * NQ * UPQ + c * UPQ + u
                    d = rcopy(out_ref.at[uid], out_ref.at[uid],
                              p3s.at[l, t, u], p3r.at[l, t, u], peer[l])
                    d.start()
                    p3descs[l][t][u] = d
        for u in range(UPQ):
            for l in range(2):
                p3descs[l][NQ - 2][u].wait_recv()

        for l in range(2):
            for s in range(NQ - 1):
                for u in range(UPQ):
                    p1descs[l][s][u].wait_send()
                    p3descs[l][s][u].wait_send()
        @pl.when(z < NZ - 1)
        def _():
            for l in range(2):
                for u in range(UPQ):
                    if u % 2 == 0:
                        p2r_descs[l][u].wait_send()
                    else:
                        p2b_descs[l][u].wait_send()
        @pl.when(z > 0)
        def _():
            for l in range(2):
                for u in range(UPQ):
                    if u % 2 == 0:
                        p2b_descs[l][u].wait_send()
                    else:
                        p2r_descs[l][u].wait_send()

    nunit = 2 * NQ * UPQ
    out = pl.pallas_call(
        body,
        out_shape=jax.ShapeDtypeStruct((nunit, u_rows, n), jnp.bfloat16),
        in_specs=[pl.BlockSpec(memory_space=pltpu.VMEM)],
        out_specs=pl.BlockSpec(memory_space=pltpu.VMEM),
        scratch_shapes=[
            pltpu.VMEM((2 * UPQ, u_rows, n), jnp.bfloat16),
            pltpu.VMEM((2 * (NQ - 1) * UPQ, u_rows, n), jnp.bfloat16),
            pltpu.VMEM((2 * UPQ, u_rows, n), jnp.bfloat16),
            pltpu.VMEM((2 * UPQ, u_rows, n), jnp.bfloat16),
            pltpu.SemaphoreType.DMA((2, NQ - 1, UPQ)),
            pltpu.SemaphoreType.DMA((2, NQ - 1, UPQ)),
            pltpu.SemaphoreType.DMA((2, UPQ)),
            pltpu.SemaphoreType.DMA((2, UPQ)),
            pltpu.SemaphoreType.DMA((2, UPQ)),
            pltpu.SemaphoreType.DMA((2, UPQ)),
            pltpu.SemaphoreType.DMA((2, NQ - 1, UPQ)),
            pltpu.SemaphoreType.DMA((2, NQ - 1, UPQ)),
        ],
        compiler_params=pltpu.CompilerParams(collective_id=0),
    )(xr)
    return out.reshape(m, n)


# device time: 121611 ns/iter; 1.3613x vs baseline; 1.0939x over previous
import jax
import jax.numpy as jnp
from jax import lax
from jax.experimental import pallas as pl
from jax.experimental.pallas import tpu as pltpu

N_DEV = 16
NSUB = 4


def kernel(x):
    m, n = x.shape
    mc = m // (2 * N_DEV)
    sub = mc // NSUB
    xr = x.reshape(2 * N_DEV, mc, n)

    def body(x_ref, out_ref, sbuf_a, sbuf_b, rbuf_a, rbuf_b,
             ssem_a, ssem_b, rsem_a, rsem_b,
             ag_ssem_a, ag_ssem_b, ag_rsem_a, ag_rsem_b):
        my = lax.axis_index("i")
        right = lax.rem(my + 1, N_DEV)
        left = lax.rem(my + N_DEV - 1, N_DEV)

        bar = pltpu.get_barrier_semaphore()
        for nbr in (left, right):
            pl.semaphore_signal(
                bar, inc=1, device_id=(nbr,),
                device_id_type=pl.DeviceIdType.MESH,
            )
        pl.semaphore_wait(bar, 2)

        rows = [pl.ds(k * sub, sub) for k in range(NSUB)]
        rs_descs = {"a": [], "b": []}
        ag_descs = {"a": [], "b": []}

        cast = lambda v: v.astype(jnp.bfloat16)

        def rs_desc(d, s, k, src):
            rbuf, rsem, ssem, peer = (
                (rbuf_a, rsem_a, ssem_a, right) if d == "a"
                else (rbuf_b, rsem_b, ssem_b, left)
            )
            return pltpu.make_async_remote_copy(
                src_ref=src,
                dst_ref=rbuf.at[s, rows[k]],
                send_sem=ssem.at[s, k],
                recv_sem=rsem.at[s, k],
                device_id=(peer,),
                device_id_type=pl.DeviceIdType.MESH,
            )

        def ag_desc(d, t, k, c):
            ssem, rsem, peer = (
                (ag_ssem_a, ag_rsem_a, right) if d == "a"
                else (ag_ssem_b, ag_rsem_b, left)
            )
            return pltpu.make_async_remote_copy(
                src_ref=out_ref.at[c, rows[k]],
                dst_ref=out_ref.at[c, rows[k]],
                send_sem=ssem.at[t, k],
                recv_sem=rsem.at[t, k],
                device_id=(peer,),
                device_id_type=pl.DeviceIdType.MESH,
            )

        for s in range(N_DEV - 1):
            ca = lax.rem(my - s + 2 * N_DEV, N_DEV)
            cb = N_DEV + lax.rem(my + s, N_DEV)
            hop_a, hop_b = [], []
            for k in range(NSUB):
                for d, c, rbuf, sbuf, hop in (
                    ("a", ca, rbuf_a, sbuf_a, hop_a),
                    ("b", cb, rbuf_b, sbuf_b, hop_b),
                ):
                    if s == 0:
                        sbuf[rows[k]] = cast(x_ref[c, rows[k]])
                        src = sbuf.at[rows[k]]
                    else:
                        rs_descs[d][s - 1][k].wait_recv()
                        if s >= 2:
                            rs_descs[d][s - 2][k].wait_send()
                        rbuf[s - 1, rows[k]] = (
                            rbuf[s - 1, rows[k]] + cast(x_ref[c, rows[k]])
                        )
                        src = rbuf.at[s - 1, rows[k]]
                    desc = rs_desc(d, s, k, src)
                    desc.start()
                    hop.append(desc)
            rs_descs["a"].append(hop_a)
            rs_descs["b"].append(hop_b)

        c_mine_a = lax.rem(my + 1, N_DEV)
        c_mine_b = N_DEV + lax.rem(my + N_DEV - 1, N_DEV)

        for t in range(N_DEV - 1):
            ca = lax.rem(my + 1 - t + 2 * N_DEV, N_DEV)
            cb = N_DEV + lax.rem(my - 1 + t + 2 * N_DEV, N_DEV)
            hop_a, hop_b = [], []
            for k in range(NSUB):
                for d, c, c_mine, rbuf, hop in (
                    ("a", ca, c_mine_a, rbuf_a, hop_a),
                    ("b", cb, c_mine_b, rbuf_b, hop_b),
                ):
                    if t == 0:
                        rs_descs[d][N_DEV - 2][k].wait_recv()
                        rs_descs[d][N_DEV - 3][k].wait_send()
                        out_ref[c_mine, rows[k]] = (
                            rbuf[N_DEV - 2, rows[k]]
                            + cast(x_ref[c_mine, rows[k]])
                        )
                    else:
                        ag_descs[d][t - 1][k].wait_recv()
                        if t == 1:
                            rs_descs[d][N_DEV - 2][k].wait_send()
                        else:
                            ag_descs[d][t - 2][k].wait_send()
                    desc = ag_desc(d, t, k, c)
                    desc.start()
                    hop.append(desc)
            ag_descs["a"].append(hop_a)
            ag_descs["b"].append(hop_b)

        for d in ("a", "b"):
            for k in range(NSUB):
                ag_descs[d][N_DEV - 2][k].wait_recv()
                ag_descs[d][N_DEV - 3][k].wait_send()
                ag_descs[d][N_DEV - 2][k].wait_send()

    out = pl.pallas_call(
        body,
        out_shape=jax.ShapeDtypeStruct((2 * N_DEV, mc, n), jnp.bfloat16),
        in_specs=[pl.BlockSpec(memory_space=pltpu.VMEM)],
        out_specs=pl.BlockSpec(memory_space=pltpu.VMEM),
        scratch_shapes=[
            pltpu.VMEM((mc, n), jnp.bfloat16),
            pltpu.VMEM((mc, n), jnp.bfloat16),
            pltpu.VMEM((N_DEV - 1, mc, n), jnp.bfloat16),
            pltpu.VMEM((N_DEV - 1, mc, n), jnp.bfloat16),
            pltpu.SemaphoreType.DMA((N_DEV - 1, NSUB)),
            pltpu.SemaphoreType.DMA((N_DEV - 1, NSUB)),
            pltpu.SemaphoreType.DMA((N_DEV - 1, NSUB)),
            pltpu.SemaphoreType.DMA((N_DEV - 1, NSUB)),
            pltpu.SemaphoreType.DMA((N_DEV - 1, NSUB)),
            pltpu.SemaphoreType.DMA((N_DEV - 1, NSUB)),
            pltpu.SemaphoreType.DMA((N_DEV - 1, NSUB)),
            pltpu.SemaphoreType.DMA((N_DEV - 1, NSUB)),
        ],
        compiler_params=pltpu.CompilerParams(collective_id=0),
    )(xr)
    return out.reshape(m, n)


# device time: 102591 ns/iter; 1.6137x vs baseline; 1.1854x over previous
import jax
import jax.numpy as jnp
from jax import lax
from jax.experimental import pallas as pl
from jax.experimental.pallas import tpu as pltpu

N_DEV = 16
NSUB = 4


def kernel(x):
    m, n = x.shape
    mc = m // (2 * N_DEV)
    sub = mc // NSUB
    xr = x.reshape(2 * N_DEV, mc, n)

    def body(x_ref, out_ref, sbuf_a, sbuf_b, rbuf_a, rbuf_b,
             ssem_a, ssem_b, rsem_a, rsem_b,
             ag_ssem_a, ag_ssem_b, ag_rsem_a, ag_rsem_b):
        my = lax.axis_index("i")
        q = lax.rem(my, 4)
        z = my // 4
        rp = jnp.where(
            q == 0, z,
            jnp.where(q == 1, 7 - z, jnp.where(q == 2, 8 + z, 15 - z)))
        right = jnp.where(
            (q == 0) | (q == 2),
            jnp.where(z < 3, my + 4, my + 1),
            jnp.where(z > 0, my - 4, jnp.where(q == 1, my + 1, my - 3)))
        left = jnp.where(
            q == 0,
            jnp.where(z > 0, my - 4, my + 3),
            jnp.where((q == 1) | (q == 3),
                      jnp.where(z < 3, my + 4, my - 1),
                      jnp.where(z > 0, my - 4, my - 1)))

        bar = pltpu.get_barrier_semaphore()
        for nbr in (left, right):
            pl.semaphore_signal(
                bar, inc=1, device_id=(nbr,),
                device_id_type=pl.DeviceIdType.MESH,
            )
        pl.semaphore_wait(bar, 2)

        rows = [pl.ds(k * sub, sub) for k in range(NSUB)]
        rs_descs = {"a": [], "b": []}
        ag_descs = {"a": [], "b": []}

        cast = lambda v: v.astype(jnp.bfloat16)

        def rs_desc(d, s, k, src):
            rbuf, rsem, ssem, peer = (
                (rbuf_a, rsem_a, ssem_a, right) if d == "a"
                else (rbuf_b, rsem_b, ssem_b, left)
            )
            return pltpu.make_async_remote_copy(
                src_ref=src,
                dst_ref=rbuf.at[s, rows[k]],
                send_sem=ssem.at[s, k],
                recv_sem=rsem.at[s, k],
                device_id=(peer,),
                device_id_type=pl.DeviceIdType.MESH,
            )

        def ag_desc(d, t, k, c):
            ssem, rsem, peer = (
                (ag_ssem_a, ag_rsem_a, right) if d == "a"
                else (ag_ssem_b, ag_rsem_b, left)
            )
            return pltpu.make_async_remote_copy(
                src_ref=out_ref.at[c, rows[k]],
                dst_ref=out_ref.at[c, rows[k]],
                send_sem=ssem.at[t, k],
                recv_sem=rsem.at[t, k],
                device_id=(peer,),
                device_id_type=pl.DeviceIdType.MESH,
            )

        for s in range(N_DEV - 1):
            ca = lax.rem(rp - s + 2 * N_DEV, N_DEV)
            cb = N_DEV + lax.rem(rp + s, N_DEV)
            hop_a, hop_b = [], []
            for k in range(NSUB):
                for d, c, rbuf, sbuf, hop in (
                    ("a", ca, rbuf_a, sbuf_a, hop_a),
                    ("b", cb, rbuf_b, sbuf_b, hop_b),
                ):
                    if s == 0:
                        sbuf[rows[k]] = cast(x_ref[c, rows[k]])
                        src = sbuf.at[rows[k]]
                    else:
                        rs_descs[d][s - 1][k].wait_recv()
                        if s >= 2:
                            rs_descs[d][s - 2][k].wait_send()
                        rbuf[s - 1, rows[k]] = (
                            rbuf[s - 1, rows[k]] + cast(x_ref[c, rows[k]])
                        )
                        src = rbuf.at[s - 1, rows[k]]
                    desc = rs_desc(d, s, k, src)
                    desc.start()
                    hop.append(desc)
            rs_descs["a"].append(hop_a)
            rs_descs["b"].append(hop_b)

        c_mine_a = lax.rem(rp + 1, N_DEV)
        c_mine_b = N_DEV + lax.rem(rp + N_DEV - 1, N_DEV)

        for t in range(N_DEV - 1):
            ca = lax.rem(rp + 1 - t + 2 * N_DEV, N_DEV)
            cb = N_DEV + lax.rem(rp - 1 + t + 2 * N_DEV, N_DEV)
            hop_a, hop_b = [], []
            for k in range(NSUB):
                for d, c, c_mine, rbuf, hop in (
                    ("a", ca, c_mine_a, rbuf_a, hop_a),
                    ("b", cb, c_mine_b, rbuf_b, hop_b),
                ):
                    if t == 0:
                        rs_descs[d][N_DEV - 2][k].wait_recv()
                        rs_descs[d][N_DEV - 3][k].wait_send()
                        out_ref[c_mine, rows[k]] = (
                            rbuf[N_DEV - 2, rows[k]]
                            + cast(x_ref[c_mine, rows[k]])
                        )
                    else:
                        ag_descs[d][t - 1][k].wait_recv()
                        if t == 1:
                            rs_descs[d][N_DEV - 2][k].wait_send()
                        else:
                            ag_descs[d][t - 2][k].wait_send()
                    desc = ag_desc(d, t, k, c)
                    desc.start()
                    hop.append(desc)
            ag_descs["a"].append(hop_a)
            ag_descs["b"].append(hop_b)

        for d in ("a", "b"):
            for k in range(NSUB):
                ag_descs[d][N_DEV - 2][k].wait_recv()
                ag_descs[d][N_DEV - 3][k].wait_send()
                ag_descs[d][N_DEV - 2][k].wait_send()

    out = pl.pallas_call(
        body,
        out_shape=jax.ShapeDtypeStruct((2 * N_DEV, mc, n), jnp.bfloat16),
        in_specs=[pl.BlockSpec(memory_space=pltpu.VMEM)],
        out_specs=pl.BlockSpec(memory_space=pltpu.VMEM),
        scratch_shapes=[
            pltpu.VMEM((mc, n), jnp.bfloat16),
            pltpu.VMEM((mc, n), jnp.bfloat16),
            pltpu.VMEM((N_DEV - 1, mc, n), jnp.bfloat16),
            pltpu.VMEM((N_DEV - 1, mc, n), jnp.bfloat16),
            pltpu.SemaphoreType.DMA((N_DEV - 1, NSUB)),
            pltpu.SemaphoreType.DMA((N_DEV - 1, NSUB)),
            pltpu.SemaphoreType.DMA((N_DEV - 1, NSUB)),
            pltpu.SemaphoreType.DMA((N_DEV - 1, NSUB)),
            pltpu.SemaphoreType.DMA((N_DEV - 1, NSUB)),
            pltpu.SemaphoreType.DMA((N_DEV - 1, NSUB)),
            pltpu.SemaphoreType.DMA((N_DEV - 1, NSUB)),
            pltpu.SemaphoreType.DMA((N_DEV - 1, NSUB)),
        ],
        compiler_params=pltpu.CompilerParams(collective_id=0),
    )(xr)
    return out.reshape(m, n)
